# baseline (speedup 1.0000x reference)
import functools

import jax
import jax.numpy as jnp
from jax import lax
from jax.experimental import pallas as pl
from jax.experimental.pallas import tpu as pltpu
from jax.experimental.pallas import tpu_sc as plsc

NUM_EMB = 1000000
DIM = 32
B = 16384

_NC = 2
_NS = 16
_NW = _NC * _NS
_BPW = B // _NW
_CH = 128
_NCH = _BPW // _CH

_mesh = plsc.VectorSubcoreMesh(core_axis_name="c", subcore_axis_name="s")


@functools.partial(
    pl.kernel,
    mesh=_mesh,
    compiler_params=pltpu.CompilerParams(
        use_tc_tiling_on_sc=False, needs_layout_passes=False),
    out_type=jax.ShapeDtypeStruct((2, B, DIM), jnp.float32),
    scratch_types=[
        pltpu.VMEM((_NCH, _CH), jnp.int32),
        pltpu.VMEM((DIM, _BPW), jnp.float32),
        pltpu.VMEM((_BPW, DIM), jnp.float32),
        pltpu.SemaphoreType.DMA,
    ],
)
def _emb_lookup(idx_hbm, wr_hbm, wi_hbm, out_hbm, idx_v, colsT, rows, sem):
    wid = lax.axis_index("s") * _NC + lax.axis_index("c")
    base = wid * _BPW
    pltpu.sync_copy(idx_hbm.at[wid], idx_v)

    lanes = lax.broadcasted_iota(jnp.int32, (16,), 0)

    for t, w_hbm in ((0, wr_hbm), (1, wi_hbm)):
        gathers = []
        for c in range(DIM):
            plane = w_hbm.at[c]
            for j in range(_NCH):
                gathers.append(pltpu.async_copy(
                    plane.at[idx_v.at[j]],
                    colsT.at[c, pl.ds(j * _CH, _CH)],
                    sem))
        for g in gathers:
            g.wait()
        def tbody(g, _):
            rvec = lanes + g * 16
            for c in range(DIM):
                col = jnp.full((16,), c, jnp.int32)
                v = plsc.load_gather(colsT, [col, rvec])
                plsc.store_scatter(rows, [rvec, col], v)
            return ()

        lax.fori_loop(0, _BPW // 16, tbody, ())
        pltpu.sync_copy(rows, out_hbm.at[t, pl.ds(base, _BPW)])


@jax.jit
def kernel(index, W_real, W_img):
    idx = index.astype(jnp.int32).reshape(_NW, _NCH, _CH)
    return _emb_lookup(idx, W_real.T, W_img.T)


# speedup vs baseline: 8.4756x; 8.4756x over previous
import functools

import jax
import jax.numpy as jnp
from jax import lax
from jax.experimental import pallas as pl
from jax.experimental.pallas import tpu as pltpu
from jax.experimental.pallas import tpu_sc as plsc

NUM_EMB = 1000000
DIM = 32
B = 16384

_NC = 2
_NS = 16
_NW = _NC * _NS
_BPW = B // _NW
_HALF = _BPW // 2

_mesh = plsc.VectorSubcoreMesh(core_axis_name="c", subcore_axis_name="s")


@functools.partial(
    pl.kernel,
    mesh=_mesh,
    out_type=jax.ShapeDtypeStruct((2, B, DIM), jnp.float32),
    scratch_types=[
        pltpu.VMEM((_BPW,), jnp.int32),
        pltpu.VMEM((_HALF, DIM), jnp.float32),
        pltpu.VMEM((_HALF, DIM), jnp.float32),
        pltpu.SemaphoreType.DMA,
    ],
)
def _emb_lookup(idx_hbm, wr_hbm, wi_hbm, out_hbm, idx_v, rows_r, rows_i, sem):
    wid = lax.axis_index("s") * _NC + lax.axis_index("c")
    base = wid * _BPW
    pltpu.sync_copy(idx_hbm.at[wid], idx_v)

    for half in range(2):
        off = half * _HALF

        def grp_body(g, _):
            grp = idx_v[pl.ds(off + g * 16, 16)]
            for lane in range(16):
                row = grp[lane]
                i = g * 16 + lane
                pltpu.async_copy(
                    wr_hbm.at[pl.ds(row, 1), :], rows_r.at[pl.ds(i, 1), :], sem)
                pltpu.async_copy(
                    wi_hbm.at[pl.ds(row, 1), :], rows_i.at[pl.ds(i, 1), :], sem)
            return ()

        lax.fori_loop(0, _HALF // 16, grp_body, ())
        pltpu.make_async_copy(
            wr_hbm.at[pl.ds(0, _HALF), :], rows_r, sem).wait()
        pltpu.make_async_copy(
            wi_hbm.at[pl.ds(0, _HALF), :], rows_i, sem).wait()
        pltpu.sync_copy(rows_r, out_hbm.at[0, pl.ds(base + off, _HALF)])
        pltpu.sync_copy(rows_i, out_hbm.at[1, pl.ds(base + off, _HALF)])


@jax.jit
def kernel(index, W_real, W_img):
    idx = index.astype(jnp.int32).reshape(_NW, _BPW)
    return _emb_lookup(idx, W_real, W_img)
